# baseline (speedup 1.0000x reference)
import functools
import math

import jax
import jax.numpy as jnp
from jax import lax
from jax.experimental import pallas as pl
from jax.experimental.pallas import tpu as pltpu
from jax.experimental.pallas import tpu_sc as plsc

FLOW_LAYERS = 3
FLOW_HID = 16
TOK_BLOCK = 2048

NC = 2
NS = 16
NW = NC * NS
LANES = 16


def _prefix_kernel(mwi_ref, didx_ref, cnt_ref):
    m = mwi_ref[...].astype(jnp.float32)
    R, C = m.shape
    rr = jax.lax.broadcasted_iota(jnp.int32, (C, C), 0)
    cc = jax.lax.broadcasted_iota(jnp.int32, (C, C), 1)
    upper = (rr <= cc).astype(jnp.float32)
    lower = (cc < rr).astype(jnp.float32)[:R, :R]
    incl = jax.lax.dot_general(m, upper, (((1,), (0,)), ((), ())),
                               preferred_element_type=jnp.float32)
    rowsum = incl[:, C - 1:C]
    rowpref = jax.lax.dot_general(lower, rowsum, (((1,), (0,)), ((), ())),
                                  preferred_element_type=jnp.float32)
    excl = rowpref + incl - m
    didx_ref[...] = jnp.where(m > 0.0, excl.astype(jnp.int32), -1)
    total = jnp.sum(m).astype(jnp.int32)
    cnt_ref[...] = jnp.full(cnt_ref.shape, total, jnp.int32)


def _widen_kernel(enc_ref, tv_ref, wide_ref):
    e = enc_ref[...]
    t = tv_ref[...]
    pad = jnp.zeros((e.shape[0], 79), jnp.float32)
    wide_ref[...] = jnp.concatenate([e, t, pad], axis=1)


def _sc_body(n_tok, n_all, didx_hbm, wide_hbm, orow_hbm,
             dv, didxb, rows, sem_g, sem_s):
    span = n_tok // NW
    nchunk = span // LANES
    w = lax.axis_index("s") * NC + lax.axis_index("c")
    base = w * span
    dump = n_all + w

    pltpu.sync_copy(didx_hbm.at[pl.ds(base, span)], dv)

    def batch_body(b, carry):
        for ci in range(nchunk):
            d = dv[pl.ds(ci * LANES, LANES)]
            didxb[ci // 8, pl.ds((ci % 8) * LANES, LANES)] = jnp.where(
                d >= 0, d + b * n_tok, dump)
        g1 = pltpu.async_copy(wide_hbm.at[pl.ds(b * n_tok + base, span)],
                              rows, sem_g)
        g1.wait()
        scs = [
            pltpu.async_copy(rows.at[pl.ds(k * 128, 128)],
                             orow_hbm.at[didxb.at[k]], sem_s)
            for k in range(span // 128)
        ]
        for s in scs:
            s.wait()
        return carry

    lax.fori_loop(0, n_all // n_tok, batch_body, jnp.int32(0))


def _flow_block(cnt_ref, blk_ref, w0t_ref, b0_ref, w1t_ref, b1_ref,
                w2t_ref, b2_ref, out_ref):
    j = pl.program_id(1)
    cnt = cnt_ref[0]
    TB = TOK_BLOCK
    jmax = jnp.maximum((cnt + TB - 1) // TB - 1, 0)

    @pl.when(j == 0)
    def _():
        out_ref[...] = jnp.zeros_like(out_ref)

    @pl.when(j <= jmax)
    def _():
        blk = blk_ref[...]
        enc = blk[:, :48].astype(jnp.bfloat16)
        lane = jax.lax.broadcasted_iota(jnp.int32, (1, 128), 1)
        sel = (lane == 48).astype(jnp.float32)
        x0 = jax.lax.dot_general(
            sel, blk, (((1,), (1,)), ((), ())),
            preferred_element_type=jnp.float32)
        h1t = jax.lax.dot_general(
            w0t_ref[...], enc, (((1,), (1,)), ((), ())),
            preferred_element_type=jnp.float32)
        h1t = jax.nn.relu(h1t + b0_ref[...]).astype(jnp.bfloat16)
        h2t = jax.lax.dot_general(
            w1t_ref[...], h1t, (((1,), (0,)), ((), ())),
            preferred_element_type=jnp.float32)
        h2t = jax.nn.relu(h2t + b1_ref[...]).astype(jnp.bfloat16)
        pt = jax.lax.dot_general(
            w2t_ref[...], h2t, (((1,), (0,)), ((), ())),
            preferred_element_type=jnp.float32)
        pt = pt + b2_ref[...]

        x = x0
        logdet = jnp.zeros(x.shape, dtype=jnp.float32)
        delta = 1e-6
        for l in range(FLOW_LAYERS):
            base = l * 3 * FLOW_HID
            ap = pt[base:base + FLOW_HID]
            bp = pt[base + FLOW_HID:base + 2 * FLOW_HID]
            wp = pt[base + 2 * FLOW_HID:base + 3 * FLOW_HID]
            e1 = jnp.exp(-jnp.abs(ap))
            a = jnp.maximum(ap, 0.0) + jnp.log1p(e1)
            la = jnp.log(a)
            mw_ = jnp.max(wp, axis=0, keepdims=True)
            ew = jnp.exp(wp - mw_)
            sew = jnp.sum(ew, axis=0, keepdims=True)
            wl = wp - mw_ - jnp.log(sew)
            pre = a * x + bp
            apre = jnp.abs(pre)
            e2 = jnp.exp(-apre)
            l2 = jnp.log1p(e2)
            r = 1.0 / (1.0 + e2)
            sig = jnp.where(pre >= 0.0, r, e2 * r)
            lsig2 = -(apre + 2.0 * l2)
            x_pre = jnp.sum(ew * sig, axis=0, keepdims=True) / sew
            arg = wl + lsig2 + la
            m2 = jnp.max(arg, axis=0, keepdims=True)
            logj = m2 + jnp.log(
                jnp.sum(jnp.exp(arg - m2), axis=0, keepdims=True))
            logdet = logdet + logj
            if l < FLOW_LAYERS - 1:
                xc = jnp.clip(x_pre, delta, 1.0 - delta)
                lxc = jnp.log(xc)
                l1m = jnp.log1p(-xc)
                x = lxc - l1m
                logdet = logdet - lxc - l1m

        pos = j * TB + jax.lax.broadcasted_iota(jnp.int32, x.shape, 1)
        partial = jnp.sum(jnp.where(pos < cnt, logdet, 0.0))
        out_ref[...] = out_ref[...] - partial


def kernel(encoded, mask, true_value, W0, b0, W1, b1, W2, b2):
    B, S, T, D = encoded.shape
    N = S * T
    NALL = B * N
    TB = TOK_BLOCK
    NT = N // TB
    NPAD = NALL + TB

    encr = encoded.reshape(NALL, D)
    tvr = true_value.reshape(NALL, 1)
    mwi2 = (~mask.reshape(B, N)[0]).astype(jnp.int32).reshape(N // 128, 128)

    didx, cnt = pl.pallas_call(
        _prefix_kernel,
        grid=(1,),
        in_specs=[pl.BlockSpec((N // 128, 128), lambda i: (0, 0))],
        out_specs=[
            pl.BlockSpec((N // 128, 128), lambda i: (0, 0)),
            pl.BlockSpec((1, 128), lambda i: (0, 0)),
        ],
        out_shape=[
            jax.ShapeDtypeStruct((N // 128, 128), jnp.int32),
            jax.ShapeDtypeStruct((1, 128), jnp.int32),
        ],
    )(mwi2)
    didx1 = didx.reshape(N)
    cnt16 = cnt.reshape(128)[:LANES]

    wide = pl.pallas_call(
        _widen_kernel,
        grid=(NALL // TB,),
        in_specs=[
            pl.BlockSpec((TB, D), lambda i: (i, 0)),
            pl.BlockSpec((TB, 1), lambda i: (i, 0)),
        ],
        out_specs=pl.BlockSpec((TB, 128), lambda i: (i, 0)),
        out_shape=jax.ShapeDtypeStruct((NALL, 128), jnp.float32),
    )(encr, tvr)

    mesh = plsc.VectorSubcoreMesh(core_axis_name="c", subcore_axis_name="s",
                                  num_cores=NC, num_subcores=NS)
    span = N // NW
    sc = pl.kernel(
        functools.partial(_sc_body, N, NALL),
        out_type=[
            jax.ShapeDtypeStruct((NPAD, 128), jnp.float32),
        ],
        mesh=mesh,
        scratch_types=[
            pltpu.VMEM((span,), jnp.int32),
            pltpu.VMEM((span // 128, 128), jnp.int32),
            pltpu.VMEM((span, 128), jnp.float32),
            pltpu.SemaphoreType.DMA,
            pltpu.SemaphoreType.DMA,
        ],
    )
    (crows,) = sc(didx1, wide)

    P = FLOW_LAYERS * 3 * FLOW_HID

    grid_spec = pltpu.PrefetchScalarGridSpec(
        num_scalar_prefetch=1,
        grid=(B, NT),
        in_specs=[
            pl.BlockSpec(
                (TB, 128),
                lambda b, j, cnt: (
                    b * NT + jnp.minimum(
                        j, jnp.maximum((cnt[0] + TB - 1) // TB - 1, 0)),
                    0)),
            pl.BlockSpec((128, D), lambda b, j, cnt: (0, 0)),
            pl.BlockSpec((128, 1), lambda b, j, cnt: (0, 0)),
            pl.BlockSpec((128, 128), lambda b, j, cnt: (0, 0)),
            pl.BlockSpec((128, 1), lambda b, j, cnt: (0, 0)),
            pl.BlockSpec((P, 128), lambda b, j, cnt: (0, 0)),
            pl.BlockSpec((P, 1), lambda b, j, cnt: (0, 0)),
        ],
        out_specs=pl.BlockSpec((1, 1, 128), lambda b, j, cnt: (b, 0, 0)),
    )
    out = pl.pallas_call(
        _flow_block,
        grid_spec=grid_spec,
        out_shape=jax.ShapeDtypeStruct((B, 1, 128), jnp.float32),
    )(cnt16, crows, W0.T.astype(jnp.bfloat16), b0.reshape(-1, 1),
      W1.T.astype(jnp.bfloat16), b1.reshape(-1, 1),
      W2.T.astype(jnp.bfloat16), b2.reshape(-1, 1))
    return out[:, 0, 0]


# speedup vs baseline: 3.3536x; 3.3536x over previous
import functools
import math

import jax
import jax.numpy as jnp
from jax.experimental import pallas as pl

FLOW_LAYERS = 3
FLOW_HID = 16
TOK_BLOCK = 4096


def _block_kernel(enc_ref, tv_ref, mw_ref, w0t_ref, b0_ref, w1t_ref, b1_ref,
                  w2t_ref, b2_ref, out_ref):
    j = pl.program_id(1)

    enc = enc_ref[0].astype(jnp.bfloat16)
    h1t = jax.lax.dot_general(
        w0t_ref[...], enc, (((1,), (1,)), ((), ())),
        preferred_element_type=jnp.float32)
    h1t = jax.nn.relu(h1t + b0_ref[...]).astype(jnp.bfloat16)
    h2t = jax.lax.dot_general(
        w1t_ref[...], h1t, (((1,), (0,)), ((), ())),
        preferred_element_type=jnp.float32)
    h2t = jax.nn.relu(h2t + b1_ref[...]).astype(jnp.bfloat16)
    pt = jax.lax.dot_general(
        w2t_ref[...], h2t, (((1,), (0,)), ((), ())),
        preferred_element_type=jnp.float32)
    pt = pt + b2_ref[...]

    x = tv_ref[0]
    logdet = jnp.zeros(x.shape, dtype=jnp.float32)
    delta = 1e-6
    for l in range(FLOW_LAYERS):
        base = l * 3 * FLOW_HID
        ap = pt[base:base + FLOW_HID]
        bp = pt[base + FLOW_HID:base + 2 * FLOW_HID]
        wp = pt[base + 2 * FLOW_HID:base + 3 * FLOW_HID]
        e1 = jnp.exp(-jnp.abs(ap))
        a = jnp.maximum(ap, 0.0) + jnp.log(1.0 + e1)
        la = jnp.log(a)
        mw_ = jnp.max(wp, axis=0, keepdims=True)
        ew = jnp.exp(wp - mw_)
        sew = jnp.sum(ew, axis=0, keepdims=True)
        wl = wp - mw_ - jnp.log(sew)
        pre = a * x + bp
        apre = jnp.abs(pre)
        e2 = jnp.exp(-apre)
        l2 = jnp.log(1.0 + e2)
        r = 1.0 / (1.0 + e2)
        sig = jnp.where(pre >= 0.0, r, e2 * r)
        lsig2 = -(apre + 2.0 * l2)
        x_pre = jnp.sum(ew * sig, axis=0, keepdims=True) / sew
        arg = wl + lsig2 + la
        m2 = jnp.max(arg, axis=0, keepdims=True)
        logj = m2 + jnp.log(jnp.sum(jnp.exp(arg - m2), axis=0, keepdims=True))
        logdet = logdet + logj
        if l < FLOW_LAYERS - 1:
            xc = jnp.clip(x_pre, delta, 1.0 - delta)
            lxc = jnp.log(xc)
            l1m = jnp.log1p(-xc)
            x = lxc - l1m
            logdet = logdet - lxc - l1m

    partial = jnp.sum(mw_ref[0] * logdet)

    @pl.when(j == 0)
    def _():
        out_ref[...] = jnp.zeros_like(out_ref)

    out_ref[...] = out_ref[...] - partial


def kernel(encoded, mask, true_value, W0, b0, W1, b1, W2, b2):
    B, S, T, D = encoded.shape
    N = S * T
    TB = TOK_BLOCK
    NT = N // TB

    enc3 = encoded.reshape(B, N, D)
    tv3 = true_value.reshape(B, 1, N)
    m0 = mask.reshape(B, N)[0]
    mw = (~m0).astype(jnp.float32).reshape(1, 1, N)
    P = FLOW_LAYERS * 3 * FLOW_HID

    out = pl.pallas_call(
        _block_kernel,
        grid=(B, NT),
        in_specs=[
            pl.BlockSpec((1, TB, D), lambda b, j: (b, j, 0)),
            pl.BlockSpec((1, 1, TB), lambda b, j: (b, 0, j)),
            pl.BlockSpec((1, 1, TB), lambda b, j: (0, 0, j)),
            pl.BlockSpec((128, D), lambda b, j: (0, 0)),
            pl.BlockSpec((128, 1), lambda b, j: (0, 0)),
            pl.BlockSpec((128, 128), lambda b, j: (0, 0)),
            pl.BlockSpec((128, 1), lambda b, j: (0, 0)),
            pl.BlockSpec((P, 128), lambda b, j: (0, 0)),
            pl.BlockSpec((P, 1), lambda b, j: (0, 0)),
        ],
        out_specs=pl.BlockSpec((1, 1, 128), lambda b, j: (b, 0, 0)),
        out_shape=jax.ShapeDtypeStruct((B, 1, 128), jnp.float32),
    )(enc3, tv3, mw, W0.T.astype(jnp.bfloat16), b0.reshape(-1, 1),
      W1.T.astype(jnp.bfloat16), b1.reshape(-1, 1),
      W2.T.astype(jnp.bfloat16), b2.reshape(-1, 1))
    return out[:, 0, 0]


# speedup vs baseline: 3.6289x; 1.0821x over previous
import functools
import math

import jax
import jax.numpy as jnp
from jax.experimental import pallas as pl

FLOW_LAYERS = 3
FLOW_HID = 16
TOK_BLOCK = 4096


def _block_kernel(enc_ref, tv_ref, mw_ref, w0t_ref, b0_ref, w1t_ref, b1_ref,
                  w2t_ref, b2_ref, out_ref):
    j = pl.program_id(1)

    enc = enc_ref[0].astype(jnp.bfloat16)
    h1t = jax.lax.dot_general(
        w0t_ref[...], enc, (((1,), (1,)), ((), ())),
        preferred_element_type=jnp.float32)
    h1t = jax.nn.relu(h1t + b0_ref[...]).astype(jnp.bfloat16)
    h2t = jax.lax.dot_general(
        w1t_ref[...], h1t, (((1,), (0,)), ((), ())),
        preferred_element_type=jnp.float32)
    h2t = jax.nn.relu(h2t + b1_ref[...]).astype(jnp.bfloat16)
    pt = jax.lax.dot_general(
        w2t_ref[...], h2t, (((1,), (0,)), ((), ())),
        preferred_element_type=jnp.float32)
    pt = pt + b2_ref[...]

    x = tv_ref[0]
    logdet = jnp.zeros(x.shape, dtype=jnp.float32)
    delta = 1e-6
    for l in range(FLOW_LAYERS):
        base = l * 3 * FLOW_HID
        ap = pt[base:base + FLOW_HID]
        bp = pt[base + FLOW_HID:base + 2 * FLOW_HID]
        wp = pt[base + 2 * FLOW_HID:base + 3 * FLOW_HID]
        e1 = jnp.exp(-jnp.abs(ap))
        a = jnp.maximum(ap, 0.0) + jnp.log(1.0 + e1)
        mw_ = jnp.max(wp, axis=0, keepdims=True)
        ew = jnp.exp(wp - mw_)
        sew = jnp.sum(ew, axis=0, keepdims=True)
        pre = a * x + bp
        e2 = jnp.exp(-jnp.abs(pre))
        r = 1.0 / (1.0 + e2)
        e2r = e2 * r
        ge = pre >= 0.0
        sig = jnp.where(ge, r, e2r)
        sigc = jnp.where(ge, e2r, r)
        x_pre = jnp.sum(ew * sig, axis=0, keepdims=True) / sew
        j_lin = jnp.sum(ew * (a * (sig * sigc)), axis=0, keepdims=True)
        logj = jnp.log(jnp.maximum(j_lin, 1e-37)) - jnp.log(sew)
        logdet = logdet + logj
        if l < FLOW_LAYERS - 1:
            xc = jnp.clip(x_pre, delta, 1.0 - delta)
            lxc = jnp.log(xc)
            l1m = jnp.log1p(-xc)
            x = lxc - l1m
            logdet = logdet - lxc - l1m

    partial = jnp.sum(mw_ref[0] * logdet)

    @pl.when(j == 0)
    def _():
        out_ref[...] = jnp.zeros_like(out_ref)

    out_ref[...] = out_ref[...] - partial


def kernel(encoded, mask, true_value, W0, b0, W1, b1, W2, b2):
    B, S, T, D = encoded.shape
    N = S * T
    TB = TOK_BLOCK
    NT = N // TB

    enc3 = encoded.reshape(B, N, D)
    tv3 = true_value.reshape(B, 1, N)
    m0 = mask.reshape(B, N)[0]
    mw = (~m0).astype(jnp.float32).reshape(1, 1, N)
    P = FLOW_LAYERS * 3 * FLOW_HID

    out = pl.pallas_call(
        _block_kernel,
        grid=(B, NT),
        in_specs=[
            pl.BlockSpec((1, TB, D), lambda b, j: (b, j, 0)),
            pl.BlockSpec((1, 1, TB), lambda b, j: (b, 0, j)),
            pl.BlockSpec((1, 1, TB), lambda b, j: (0, 0, j)),
            pl.BlockSpec((128, D), lambda b, j: (0, 0)),
            pl.BlockSpec((128, 1), lambda b, j: (0, 0)),
            pl.BlockSpec((128, 128), lambda b, j: (0, 0)),
            pl.BlockSpec((128, 1), lambda b, j: (0, 0)),
            pl.BlockSpec((P, 128), lambda b, j: (0, 0)),
            pl.BlockSpec((P, 1), lambda b, j: (0, 0)),
        ],
        out_specs=pl.BlockSpec((1, 1, 128), lambda b, j: (b, 0, 0)),
        out_shape=jax.ShapeDtypeStruct((B, 1, 128), jnp.float32),
    )(enc3, tv3, mw, W0.T.astype(jnp.bfloat16), b0.reshape(-1, 1),
      W1.T.astype(jnp.bfloat16), b1.reshape(-1, 1),
      W2.T.astype(jnp.bfloat16), b2.reshape(-1, 1))
    return out[:, 0, 0]


# speedup vs baseline: 3.8496x; 1.0608x over previous
import functools
import math

import jax
import jax.numpy as jnp
from jax.experimental import pallas as pl

FLOW_LAYERS = 3
FLOW_HID = 16
TOK_BLOCK = 8192


def _block_kernel(enc_ref, tv_ref, mw_ref, w0t_ref, b0_ref, w1t_ref, b1_ref,
                  w2t_ref, b2_ref, out_ref):
    j = pl.program_id(1)

    enc = enc_ref[0].astype(jnp.bfloat16)
    h1t = jax.lax.dot_general(
        w0t_ref[...], enc, (((1,), (1,)), ((), ())),
        preferred_element_type=jnp.float32)
    h1t = jax.nn.relu(h1t + b0_ref[...]).astype(jnp.bfloat16)
    h2t = jax.lax.dot_general(
        w1t_ref[...], h1t, (((1,), (0,)), ((), ())),
        preferred_element_type=jnp.float32)
    h2t = jax.nn.relu(h2t + b1_ref[...]).astype(jnp.bfloat16)
    pt = jax.lax.dot_general(
        w2t_ref[...], h2t, (((1,), (0,)), ((), ())),
        preferred_element_type=jnp.float32)
    pt = pt + b2_ref[...]

    x = tv_ref[0]
    logdet = jnp.zeros(x.shape, dtype=jnp.float32)
    delta = 1e-6
    for l in range(FLOW_LAYERS):
        base = l * 3 * FLOW_HID
        ap = pt[base:base + FLOW_HID]
        bp = pt[base + FLOW_HID:base + 2 * FLOW_HID]
        wp = pt[base + 2 * FLOW_HID:base + 3 * FLOW_HID]
        e1 = jnp.exp(-jnp.abs(ap))
        a = jnp.maximum(ap, 0.0) + jnp.log(1.0 + e1)
        ew = jnp.exp(wp)
        sew = jnp.sum(ew, axis=0, keepdims=True)
        pre = a * x + bp
        e2 = jnp.exp(-jnp.abs(pre))
        r = 1.0 / (1.0 + e2)
        e2r = e2 * r
        ge = pre >= 0.0
        sig = jnp.where(ge, r, e2r)
        sigc = jnp.where(ge, e2r, r)
        ews = ew * sig
        x_pre = jnp.sum(ews, axis=0, keepdims=True) / sew
        j_lin = jnp.sum(ews * (a * sigc), axis=0, keepdims=True)
        logj = jnp.log(jnp.maximum(j_lin, 1e-37)) - jnp.log(sew)
        logdet = logdet + logj
        if l < FLOW_LAYERS - 1:
            xc = jnp.clip(x_pre, delta, 1.0 - delta)
            lxc = jnp.log(xc)
            l1m = jnp.log1p(-xc)
            x = lxc - l1m
            logdet = logdet - lxc - l1m

    partial = jnp.sum(mw_ref[0] * logdet)

    @pl.when(j == 0)
    def _():
        out_ref[...] = jnp.zeros_like(out_ref)

    out_ref[...] = out_ref[...] - partial


def kernel(encoded, mask, true_value, W0, b0, W1, b1, W2, b2):
    B, S, T, D = encoded.shape
    N = S * T
    TB = TOK_BLOCK
    NT = N // TB

    enc3 = encoded.reshape(B, N, D)
    tv3 = true_value.reshape(B, 1, N)
    m0 = mask.reshape(B, N)[0]
    mw = (~m0).astype(jnp.float32).reshape(1, 1, N)
    P = FLOW_LAYERS * 3 * FLOW_HID

    out = pl.pallas_call(
        _block_kernel,
        grid=(B, NT),
        in_specs=[
            pl.BlockSpec((1, TB, D), lambda b, j: (b, j, 0)),
            pl.BlockSpec((1, 1, TB), lambda b, j: (b, 0, j)),
            pl.BlockSpec((1, 1, TB), lambda b, j: (0, 0, j)),
            pl.BlockSpec((128, D), lambda b, j: (0, 0)),
            pl.BlockSpec((128, 1), lambda b, j: (0, 0)),
            pl.BlockSpec((128, 128), lambda b, j: (0, 0)),
            pl.BlockSpec((128, 1), lambda b, j: (0, 0)),
            pl.BlockSpec((P, 128), lambda b, j: (0, 0)),
            pl.BlockSpec((P, 1), lambda b, j: (0, 0)),
        ],
        out_specs=pl.BlockSpec((1, 1, 128), lambda b, j: (b, 0, 0)),
        out_shape=jax.ShapeDtypeStruct((B, 1, 128), jnp.float32),
    )(enc3, tv3, mw, W0.T.astype(jnp.bfloat16), b0.reshape(-1, 1),
      W1.T.astype(jnp.bfloat16), b1.reshape(-1, 1),
      W2.T.astype(jnp.bfloat16), b2.reshape(-1, 1))
    return out[:, 0, 0]
